# baseline (speedup 1.0000x reference)
import functools

import jax
import jax.numpy as jnp
from jax.experimental import pallas as pl
from jax.experimental.pallas import tpu as pltpu

B, P, L, M, D = 8, 512, 2048, 256, 1024
PH = P // 2
LH = L // 2


def _fused_kernel(head_ref, tail_ref, x_ref, cmp_ref, lens_ref, lbl_ref,
                  mask_ref, w1_ref, b1_ref, w2_ref, b2_ref,
                  out_ref, acc_ref, coref_ref, w1b_ref, xb_ref):
    b = pl.program_id(0)
    q = pl.program_id(1)

    f32 = jnp.float32
    bf16 = jnp.bfloat16
    dot = functools.partial(jax.lax.dot_general, preferred_element_type=f32)

    @pl.when(jnp.logical_and(b == 0, q == 0))
    def _():
        w1b_ref[...] = w1_ref[...].astype(bf16)

    @pl.when(q == 0)
    def _():
        xb_ref[...] = x_ref[0].astype(bf16)

    xb = xb_ref[...]
    hr = dot(head_ref[0].astype(bf16), xb, (((1,), (0,)), ((), ())))
    tr = dot(tail_ref[0].astype(bf16), xb, (((1,), (0,)), ((), ())))

    h = (dot(hr.astype(bf16), w1b_ref[0:D], (((1,), (0,)), ((), ())))
         + dot(tr.astype(bf16), w1b_ref[D:2 * D], (((1,), (0,)), ((), ())))
         + dot((hr * tr).astype(bf16), w1b_ref[2 * D:3 * D],
               (((1,), (0,)), ((), ())))
         + b1_ref[...])
    h = jnp.maximum(h, 0.0)
    logits = (dot(h.astype(bf16), w2_ref[...], (((1,), (1,)), ((), ())))
              + b2_ref[...])

    l0 = logits[:, 0:1]
    l1 = logits[:, 1:2]
    mx = jnp.maximum(l0, l1)
    lse = mx + jnp.log(jnp.exp(l0 - mx) + jnp.exp(l1 - mx))
    logq = logits - lse
    lbl = lbl_ref[0]
    pos = lbl > 0.0
    pw = jnp.where(pos, lbl * (jnp.log(jnp.where(pos, lbl, 1.0)) - logq), 0.0)
    mask_row = mask_ref[0]
    pw_rows = pw[:, 0:1] + pw[:, 1:2]
    msum = dot(mask_row, pw_rows, (((1,), (0,)), ((), ())))
    mcnt = jnp.sum(mask_row)

    @pl.when(q == 0)
    def _():
        lens_col = lens_ref[0]
        row_i = jax.lax.broadcasted_iota(jnp.int32, (M, M), 0)
        col_j = jax.lax.broadcasted_iota(jnp.int32, (M, M), 1)
        ltri = (row_i > col_j).astype(f32)
        off = dot(ltri, lens_col, (((1,), (0,)), ((), ())))
        off_i = off.astype(jnp.int32)
        valid = lens_col > 0.0
        iota_p = jax.lax.broadcasted_iota(jnp.int32, (M, PH), 1)
        sel = (iota_p == off_i).astype(f32)
        gathered = dot(sel, l1, (((1,), (0,)), ((), ())))
        w = jnp.where(valid, gathered, 0.0)
        coref_ref[...] = dot((sel * w).astype(bf16), tr.astype(bf16),
                             (((1,), (0,)), ((), ())))

    xs = x_ref[0, pl.ds(q * LH, LH), :]
    enc = xs + dot(cmp_ref[0].astype(bf16), coref_ref[...].astype(bf16),
                   (((0,), (0,)), ((), ())))
    out_ref[0] = enc

    first = jnp.logical_and(b == 0, q == 0)
    prev = jnp.where(first, 0.0, acc_ref[...])
    upd = jnp.concatenate([msum, mcnt[None, None]], axis=1)
    acc_ref[...] = prev + upd


def kernel(head, tail, lens, input, coref_mention_position, coref_label,
           coref_label_mask, W1, b1, W2, b2):
    bf16 = jnp.bfloat16
    lens_col = lens.astype(jnp.float32).reshape(B, M, 1)
    mask_row = coref_label_mask.astype(jnp.float32).reshape(B, 1, P)
    b1r = b1.reshape(1, D)
    b2r = b2.reshape(1, 2)
    W2b = W2.T.astype(jnp.bfloat16)

    encoded, acc = pl.pallas_call(
        _fused_kernel,
        grid=(B, 2),
        in_specs=[
            pl.BlockSpec((1, PH, L), lambda b, q: (b, q, 0)),
            pl.BlockSpec((1, PH, L), lambda b, q: (b, q, 0)),
            pl.BlockSpec((1, L, D), lambda b, q: (b, 0, 0)),
            pl.BlockSpec((1, M, LH), lambda b, q: (b, 0, q)),
            pl.BlockSpec((1, M, 1), lambda b, q: (b, 0, 0)),
            pl.BlockSpec((1, PH, 2), lambda b, q: (b, q, 0)),
            pl.BlockSpec((1, 1, PH), lambda b, q: (b, 0, q)),
            pl.BlockSpec((3 * D, D), lambda b, q: (0, 0)),
            pl.BlockSpec((1, D), lambda b, q: (0, 0)),
            pl.BlockSpec((2, D), lambda b, q: (0, 0)),
            pl.BlockSpec((1, 2), lambda b, q: (0, 0)),
        ],
        out_specs=[
            pl.BlockSpec((1, LH, D), lambda b, q: (b, q, 0)),
            pl.BlockSpec((1, 2), lambda b, q: (0, 0)),
        ],
        out_shape=[
            jax.ShapeDtypeStruct((B, L, D), jnp.float32),
            jax.ShapeDtypeStruct((1, 2), jnp.float32),
        ],
        scratch_shapes=[pltpu.VMEM((M, D), jnp.float32),
                        pltpu.VMEM((3 * D, D), jnp.bfloat16),
                        pltpu.VMEM((L, D), jnp.bfloat16)],
        compiler_params=pltpu.CompilerParams(
            vmem_limit_bytes=100 * 1024 * 1024),
    )(head, tail, input, coref_mention_position, lens_col, coref_label,
      mask_row, W1, b1r, W2b, b2r)

    loss = acc[0, 0] / (2.0 * acc[0, 1])
    return (encoded, loss)


# speedup vs baseline: 1.0621x; 1.0621x over previous
import functools

import jax
import jax.numpy as jnp
from jax.experimental import pallas as pl
from jax.experimental.pallas import tpu as pltpu

B, P, L, M, D = 8, 512, 2048, 256, 1024
PH = P // 2
LH = L // 2


def _fused_kernel(head_ref, tail_ref, x_ref, cmp_ref, lens_ref, lbl_ref,
                  mask_ref, w1_ref, b1_ref, w2_ref, b2_ref,
                  out_ref, acc_ref, coref_ref, w1b_ref):
    b = pl.program_id(0)
    q = pl.program_id(1)

    f32 = jnp.float32
    bf16 = jnp.bfloat16
    dot = functools.partial(jax.lax.dot_general, preferred_element_type=f32)

    @pl.when(jnp.logical_and(b == 0, q == 0))
    def _():
        w1b_ref[...] = w1_ref[...].astype(bf16)

    nk = 4
    kc = L // nk
    hr = jnp.zeros((PH, D), f32)
    tr = jnp.zeros((PH, D), f32)
    for k in range(nk):
        xk = x_ref[0, k * kc:(k + 1) * kc, :].astype(bf16)
        hk = head_ref[0][:, k * kc:(k + 1) * kc].astype(bf16)
        tk = tail_ref[0][:, k * kc:(k + 1) * kc].astype(bf16)
        hr = hr + dot(hk, xk, (((1,), (0,)), ((), ())))
        tr = tr + dot(tk, xk, (((1,), (0,)), ((), ())))

    h = (dot(hr.astype(bf16), w1b_ref[0:D], (((1,), (0,)), ((), ())))
         + dot(tr.astype(bf16), w1b_ref[D:2 * D], (((1,), (0,)), ((), ())))
         + dot((hr * tr).astype(bf16), w1b_ref[2 * D:3 * D],
               (((1,), (0,)), ((), ())))
         + b1_ref[...])
    h = jnp.maximum(h, 0.0)
    hb = h.astype(bf16)

    logits_t = (dot(w2_ref[...], hb, (((1,), (1,)), ((), ())))
                + b2_ref[...])
    l0 = logits_t[0:1, :]
    l1 = logits_t[1:2, :]
    mx = jnp.maximum(l0, l1)
    lse = mx + jnp.log(jnp.exp(l0 - mx) + jnp.exp(l1 - mx))
    logq = logits_t - lse
    lbl = lbl_ref[0]
    pos = lbl > 0.0
    pw = jnp.where(pos, lbl * (jnp.log(jnp.where(pos, lbl, 1.0)) - logq), 0.0)
    mask_row = mask_ref[0]
    msum = jnp.sum(pw * mask_row)
    mcnt = jnp.sum(mask_row)

    @pl.when(q == 0)
    def _():
        lens_col = lens_ref[0]
        row_i = jax.lax.broadcasted_iota(jnp.int32, (M, M), 0)
        col_j = jax.lax.broadcasted_iota(jnp.int32, (M, M), 1)
        ltri = (row_i > col_j).astype(f32)
        off = dot(ltri, lens_col, (((1,), (0,)), ((), ())))
        off_i = off.astype(jnp.int32)
        valid = lens_col > 0.0
        iota_p = jax.lax.broadcasted_iota(jnp.int32, (M, PH), 1)
        sel = (iota_p == off_i).astype(f32)
        gathered = dot(sel, l1, (((1,), (1,)), ((), ())))
        w = jnp.where(valid, gathered, 0.0)
        coref_ref[...] = dot((sel * w).astype(bf16), tr.astype(bf16),
                             (((1,), (0,)), ((), ())))

    xs = x_ref[0, pl.ds(q * LH, LH), :]
    enc = xs + dot(cmp_ref[0].astype(bf16), coref_ref[...].astype(bf16),
                   (((0,), (0,)), ((), ())))
    out_ref[0] = enc

    first = jnp.logical_and(b == 0, q == 0)
    prev = jnp.where(first, 0.0, acc_ref[...])
    upd = jnp.concatenate([msum[None, None], mcnt[None, None]], axis=1)
    acc_ref[...] = prev + upd


def kernel(head, tail, lens, input, coref_mention_position, coref_label,
           coref_label_mask, W1, b1, W2, b2):
    lens_col = lens.astype(jnp.float32).reshape(B, M, 1)
    mask_row = coref_label_mask.astype(jnp.float32).reshape(B, 1, P)
    lbl_t = coref_label.transpose(0, 2, 1)
    b1r = b1.reshape(1, D)
    b2r = b2.reshape(2, 1)
    W2b = W2.T.astype(jnp.bfloat16)

    encoded, acc = pl.pallas_call(
        _fused_kernel,
        grid=(B, 2),
        in_specs=[
            pl.BlockSpec((1, PH, L), lambda b, q: (b, q, 0)),
            pl.BlockSpec((1, PH, L), lambda b, q: (b, q, 0)),
            pl.BlockSpec((1, L, D), lambda b, q: (b, 0, 0)),
            pl.BlockSpec((1, M, LH), lambda b, q: (b, 0, q)),
            pl.BlockSpec((1, M, 1), lambda b, q: (b, 0, 0)),
            pl.BlockSpec((1, 2, PH), lambda b, q: (b, 0, q)),
            pl.BlockSpec((1, 1, PH), lambda b, q: (b, 0, q)),
            pl.BlockSpec((3 * D, D), lambda b, q: (0, 0)),
            pl.BlockSpec((1, D), lambda b, q: (0, 0)),
            pl.BlockSpec((2, D), lambda b, q: (0, 0)),
            pl.BlockSpec((2, 1), lambda b, q: (0, 0)),
        ],
        out_specs=[
            pl.BlockSpec((1, LH, D), lambda b, q: (b, q, 0)),
            pl.BlockSpec((1, 2), lambda b, q: (0, 0)),
        ],
        out_shape=[
            jax.ShapeDtypeStruct((B, L, D), jnp.float32),
            jax.ShapeDtypeStruct((1, 2), jnp.float32),
        ],
        scratch_shapes=[pltpu.VMEM((M, D), jnp.float32),
                        pltpu.VMEM((3 * D, D), jnp.bfloat16)],
        compiler_params=pltpu.CompilerParams(
            vmem_limit_bytes=100 * 1024 * 1024),
    )(head, tail, input, coref_mention_position, lens_col, lbl_t,
      mask_row, W1, b1r, W2b, b2r)

    loss = acc[0, 0] / (2.0 * acc[0, 1])
    return (encoded, loss)


# speedup vs baseline: 1.0814x; 1.0182x over previous
import functools

import jax
import jax.numpy as jnp
from jax.experimental import pallas as pl
from jax.experimental.pallas import tpu as pltpu

B, P, L, M, D = 8, 512, 2048, 256, 1024
PH = P // 2
LH = L // 2


def _fused_kernel(head_ref, tail_ref, x_ref, cmp_ref, lens_ref, lbl_ref,
                  mask_ref, w1_ref, b1_ref, w2_ref, b2_ref,
                  out_ref, acc_ref, coref_ref):
    b = pl.program_id(0)
    q = pl.program_id(1)

    f32 = jnp.float32
    bf16 = jnp.bfloat16
    dot = functools.partial(jax.lax.dot_general, preferred_element_type=f32)

    x = x_ref[0]
    hr = dot(head_ref[0], x, (((1,), (0,)), ((), ())))
    tr = dot(tail_ref[0], x, (((1,), (0,)), ((), ())))

    h = (dot(hr, w1_ref[0:D], (((1,), (0,)), ((), ())))
         + dot(tr, w1_ref[D:2 * D], (((1,), (0,)), ((), ())))
         + dot(hr * tr, w1_ref[2 * D:3 * D], (((1,), (0,)), ((), ())))
         + b1_ref[...])
    h = jnp.maximum(h, 0.0)
    hb = h

    logits_t = (dot(w2_ref[...], hb, (((1,), (1,)), ((), ())))
                + b2_ref[...])
    l0 = logits_t[0:1, :]
    l1 = logits_t[1:2, :]
    mx = jnp.maximum(l0, l1)
    lse = mx + jnp.log(jnp.exp(l0 - mx) + jnp.exp(l1 - mx))
    logq = logits_t - lse
    lbl = lbl_ref[0]
    pos = lbl > 0.0
    pw = jnp.where(pos, lbl * (jnp.log(jnp.where(pos, lbl, 1.0)) - logq), 0.0)
    mask_row = mask_ref[0]
    msum = jnp.sum(pw * mask_row)
    mcnt = jnp.sum(mask_row)

    @pl.when(q == 0)
    def _():
        lens_col = lens_ref[0]
        row_i = jax.lax.broadcasted_iota(jnp.int32, (M, M), 0)
        col_j = jax.lax.broadcasted_iota(jnp.int32, (M, M), 1)
        ltri = (row_i > col_j).astype(f32)
        off = dot(ltri, lens_col, (((1,), (0,)), ((), ())))
        off_i = off.astype(jnp.int32)
        valid = lens_col > 0.0
        iota_p = jax.lax.broadcasted_iota(jnp.int32, (M, PH), 1)
        sel = (iota_p == off_i).astype(f32)
        gathered = dot(sel, l1, (((1,), (1,)), ((), ())))
        w = jnp.where(valid, gathered, 0.0)
        coref_ref[...] = dot(sel * w, tr, (((1,), (0,)), ((), ())))

    xs = x_ref[0, pl.ds(q * LH, LH), :]
    enc = xs + dot(cmp_ref[0], coref_ref[...], (((0,), (0,)), ((), ())))
    out_ref[0] = enc

    first = jnp.logical_and(b == 0, q == 0)
    prev = jnp.where(first, 0.0, acc_ref[...])
    upd = jnp.concatenate([msum[None, None], mcnt[None, None]], axis=1)
    acc_ref[...] = prev + upd


def kernel(head, tail, lens, input, coref_mention_position, coref_label,
           coref_label_mask, W1, b1, W2, b2):
    lens_col = lens.astype(jnp.float32).reshape(B, M, 1)
    mask_row = coref_label_mask.astype(jnp.float32).reshape(B, 1, P)
    lbl_t = coref_label.transpose(0, 2, 1)
    b1r = b1.reshape(1, D)
    b2r = b2.reshape(2, 1)
    W2b = W2.T

    encoded, acc = pl.pallas_call(
        _fused_kernel,
        grid=(B, 2),
        in_specs=[
            pl.BlockSpec((1, PH, L), lambda b, q: (b, q, 0)),
            pl.BlockSpec((1, PH, L), lambda b, q: (b, q, 0)),
            pl.BlockSpec((1, L, D), lambda b, q: (b, 0, 0)),
            pl.BlockSpec((1, M, LH), lambda b, q: (b, 0, q)),
            pl.BlockSpec((1, M, 1), lambda b, q: (b, 0, 0)),
            pl.BlockSpec((1, 2, PH), lambda b, q: (b, 0, q)),
            pl.BlockSpec((1, 1, PH), lambda b, q: (b, 0, q)),
            pl.BlockSpec((3 * D, D), lambda b, q: (0, 0)),
            pl.BlockSpec((1, D), lambda b, q: (0, 0)),
            pl.BlockSpec((2, D), lambda b, q: (0, 0)),
            pl.BlockSpec((2, 1), lambda b, q: (0, 0)),
        ],
        out_specs=[
            pl.BlockSpec((1, LH, D), lambda b, q: (b, q, 0)),
            pl.BlockSpec((1, 2), lambda b, q: (0, 0)),
        ],
        out_shape=[
            jax.ShapeDtypeStruct((B, L, D), jnp.float32),
            jax.ShapeDtypeStruct((1, 2), jnp.float32),
        ],
        scratch_shapes=[pltpu.VMEM((M, D), jnp.float32)],
        compiler_params=pltpu.CompilerParams(
            vmem_limit_bytes=100 * 1024 * 1024),
    )(head, tail, input, coref_mention_position, lens_col, lbl_t,
      mask_row, W1, b1r, W2b, b2r)

    loss = acc[0, 0] / (2.0 * acc[0, 1])
    return (encoded, loss)


# speedup vs baseline: 1.1052x; 1.0220x over previous
import functools

import jax
import jax.numpy as jnp
from jax.experimental import pallas as pl
from jax.experimental.pallas import tpu as pltpu

B, P, L, M, D = 8, 512, 2048, 256, 1024
PH = P // 2
LH = L // 2


def _fused_kernel(head_ref, tail_ref, x_ref, cmp_ref, lens_ref, lbl_ref,
                  mask_ref, w1_ref, b1_ref, w2_ref, b2_ref,
                  out_ref, acc_ref, coref_ref):
    b = pl.program_id(0)
    q = pl.program_id(1)

    f32 = jnp.float32
    bf16 = jnp.bfloat16
    dot = functools.partial(jax.lax.dot_general, preferred_element_type=f32)

    x = x_ref[0]
    hr = dot(head_ref[0], x, (((1,), (0,)), ((), ())))
    tr = dot(tail_ref[0], x, (((1,), (0,)), ((), ())))

    h = (dot(hr, w1_ref[0:D], (((1,), (0,)), ((), ())))
         + dot(tr, w1_ref[D:2 * D], (((1,), (0,)), ((), ())))
         + dot(hr * tr, w1_ref[2 * D:3 * D], (((1,), (0,)), ((), ())))
         + b1_ref[...])
    h = jnp.maximum(h, 0.0)
    hb = h

    logits_t = (dot(w2_ref[...], hb, (((1,), (1,)), ((), ())))
                + b2_ref[...])
    l0 = logits_t[0:1, :]
    l1 = logits_t[1:2, :]
    mx = jnp.maximum(l0, l1)
    lse = mx + jnp.log(jnp.exp(l0 - mx) + jnp.exp(l1 - mx))
    logq = logits_t - lse
    lbl = lbl_ref[0]
    pos = lbl > 0.0
    pw = jnp.where(pos, lbl * (jnp.log(jnp.where(pos, lbl, 1.0)) - logq), 0.0)
    mask_row = mask_ref[0]
    msum = jnp.sum(pw * mask_row)
    mcnt = jnp.sum(mask_row)

    lens_col = lens_ref[0]
    row_i = jax.lax.broadcasted_iota(jnp.int32, (M, M), 0)
    col_j = jax.lax.broadcasted_iota(jnp.int32, (M, M), 1)
    ltri = (row_i > col_j).astype(f32)
    off = dot(ltri, lens_col, (((1,), (0,)), ((), ())))
    off_i = off.astype(jnp.int32)
    valid = lens_col > 0.0
    iota_p = jax.lax.broadcasted_iota(jnp.int32, (M, PH), 1)
    sel = (iota_p == off_i).astype(f32)
    gathered = dot(sel, l1, (((1,), (1,)), ((), ())))
    w = jnp.where(valid, gathered, 0.0)

    @pl.when(q == 0)
    def _():
        coref_ref[...] = dot(sel * w, tr, (((1,), (0,)), ((), ())))

    xs = x_ref[0, pl.ds(q * LH, LH), :]
    enc = xs + dot(cmp_ref[0], coref_ref[...], (((0,), (0,)), ((), ())))
    out_ref[0] = enc

    first = jnp.logical_and(b == 0, q == 0)
    prev = jnp.where(first, 0.0, acc_ref[...])
    upd = jnp.concatenate([msum[None, None], mcnt[None, None]], axis=1)
    acc_ref[...] = prev + upd


def kernel(head, tail, lens, input, coref_mention_position, coref_label,
           coref_label_mask, W1, b1, W2, b2):
    lens_col = lens.astype(jnp.float32).reshape(B, M, 1)
    mask_row = coref_label_mask.astype(jnp.float32).reshape(B, 1, P)
    lbl_t = coref_label.transpose(0, 2, 1)
    b1r = b1.reshape(1, D)
    b2r = b2.reshape(2, 1)
    W2b = W2.T

    encoded, acc = pl.pallas_call(
        _fused_kernel,
        grid=(B, 2),
        in_specs=[
            pl.BlockSpec((1, PH, L), lambda b, q: (b, q, 0)),
            pl.BlockSpec((1, PH, L), lambda b, q: (b, q, 0)),
            pl.BlockSpec((1, L, D), lambda b, q: (b, 0, 0)),
            pl.BlockSpec((1, M, LH), lambda b, q: (b, 0, q)),
            pl.BlockSpec((1, M, 1), lambda b, q: (b, 0, 0)),
            pl.BlockSpec((1, 2, PH), lambda b, q: (b, 0, q)),
            pl.BlockSpec((1, 1, PH), lambda b, q: (b, 0, q)),
            pl.BlockSpec((3 * D, D), lambda b, q: (0, 0)),
            pl.BlockSpec((1, D), lambda b, q: (0, 0)),
            pl.BlockSpec((2, D), lambda b, q: (0, 0)),
            pl.BlockSpec((2, 1), lambda b, q: (0, 0)),
        ],
        out_specs=[
            pl.BlockSpec((1, LH, D), lambda b, q: (b, q, 0)),
            pl.BlockSpec((1, 2), lambda b, q: (0, 0)),
        ],
        out_shape=[
            jax.ShapeDtypeStruct((B, L, D), jnp.float32),
            jax.ShapeDtypeStruct((1, 2), jnp.float32),
        ],
        scratch_shapes=[pltpu.VMEM((M, D), jnp.float32)],
        compiler_params=pltpu.CompilerParams(
            vmem_limit_bytes=100 * 1024 * 1024),
    )(head, tail, input, coref_mention_position, lens_col, lbl_t,
      mask_row, W1, b1r, W2b, b2r)

    loss = acc[0, 0] / (2.0 * acc[0, 1])
    return (encoded, loss)
